# baseline (speedup 1.0000x reference)
import functools

import jax
import jax.numpy as jnp
from jax import lax
from jax.experimental import pallas as pl
from jax.experimental.pallas import tpu as pltpu
from jax.experimental.pallas import tpu_sc as plsc

HID = 768
EPS = 1e-12
NC, NS, L = 2, 16, 16
NW = NC * NS
NSL = HID // L
CH = 32


def _rsqrt_vec(x):
    i = plsc.bitcast(x, jnp.int32)
    i = jnp.full((L,), 0x5F3759DF, jnp.int32) - (i >> 1)
    y = plsc.bitcast(i, jnp.float32)
    for _ in range(3):
        y = y * (1.5 - 0.5 * x * y * y)
    return y


def _body(ids_hbm, tt_hbm, at_hbm, word_hbm, pos_hbm, type_hbm, tag_hbm,
          gam_hbm, bet_hbm, out_hbm,
          idx_s, tt_s, at_s, wrows, prows, type_v, tag_v, combo_v,
          gam_v, bet_v, sem):
    tok = ids_hbm.shape[0]
    tpw = tok // NW
    g_cnt = tpw // CH
    seq = pos_hbm.shape[0]

    wid = lax.axis_index("s") * NC + lax.axis_index("c")
    base = wid * tpw
    pos0 = lax.rem(base, seq)

    pltpu.async_copy(gam_hbm, gam_v, sem).wait()
    pltpu.async_copy(bet_hbm, bet_v, sem).wait()
    pltpu.async_copy(type_hbm, type_v, sem).wait()
    pltpu.async_copy(tag_hbm, tag_v, sem).wait()
    n_tag = tag_v.shape[0]
    for t in range(type_v.shape[0]):
        for a in range(n_tag):
            for j in range(NSL):
                js = pl.ds((t * n_tag + a) * HID + j * L, L)
                combo_v[js] = type_v[t, pl.ds(j * L, L)] + tag_v[a, pl.ds(j * L, L)]

    def do_chunk(g, _):
        tb = pl.multiple_of(base + g * CH, CH)
        pb = pl.multiple_of(pos0 + g * CH, CH)
        pltpu.async_copy(ids_hbm.at[pl.ds(tb, CH)], idx_s, sem).wait()
        pltpu.async_copy(tt_hbm.at[pl.ds(tb, CH)], tt_s, sem).wait()
        pltpu.async_copy(at_hbm.at[pl.ds(tb, CH)], at_s, sem).wait()
        pltpu.async_copy(word_hbm.at[idx_s], wrows, sem).wait()
        pltpu.async_copy(pos_hbm.at[pl.ds(pb, CH)], prows, sem).wait()

        lanes = lax.iota(jnp.int32, L)

        def do_group(k, _):
            ttv = tt_s[pl.ds(k * L, L)]
            atv = at_s[pl.ds(k * L, L)]
            cvec = ttv * tag_v.shape[0] + atv

            def do_row(r16, _):
                r = k * L + r16
                rsplat = jnp.full((L,), r16, jnp.int32)
                csplat = cvec.at[rsplat].get(mode="promise_in_bounds")
                cbase = csplat * HID + lanes

                def acc_j(j, carry):
                    s1, s2 = carry
                    js = pl.ds(j * L, L)
                    cj = plsc.load_gather(combo_v, [cbase + j * L])
                    a = wrows[r, js] + prows[r, js] + cj
                    wrows[r, js] = a
                    return s1 + a, s2 + a * a

                s1, s2 = lax.fori_loop(
                    0, NSL, acc_j,
                    (jnp.zeros((L,), jnp.float32),
                     jnp.zeros((L,), jnp.float32)))
                tot = jnp.sum(s1)
                tot2 = jnp.sum(s2)
                mean = tot * (1.0 / HID)
                var = tot2 * (1.0 / HID) - mean * mean
                mvec = jnp.full((L,), mean, jnp.float32)
                rvec = _rsqrt_vec(jnp.full((L,), var + EPS, jnp.float32))

                def norm_j(j, _):
                    js = pl.ds(j * L, L)
                    a = wrows[r, js]
                    wrows[r, js] = (a - mvec) * rvec * gam_v[js] + bet_v[js]
                    return 0

                lax.fori_loop(0, NSL, norm_j, 0)
                return 0

            lax.fori_loop(0, L, do_row, 0)
            return 0

        lax.fori_loop(0, CH // L, do_group, 0)
        pltpu.async_copy(wrows, out_hbm.at[pl.ds(tb, CH)], sem).wait()
        return 0

    lax.fori_loop(0, g_cnt, do_chunk, 0)


@jax.jit
def _run(ids, tt, at, word_emb, pos_emb, type_emb, tag_emb, gamma, beta):
    tok = ids.shape[0]
    mesh = plsc.VectorSubcoreMesh(core_axis_name="c", subcore_axis_name="s")
    k = pl.kernel(
        _body,
        out_type=jax.ShapeDtypeStruct((tok, HID), jnp.float32),
        mesh=mesh,
        compiler_params=pltpu.CompilerParams(needs_layout_passes=False),
        scratch_types=[
            pltpu.VMEM((CH,), jnp.int32),
            pltpu.VMEM((CH,), jnp.int32),
            pltpu.VMEM((CH,), jnp.int32),
            pltpu.VMEM((CH, HID), jnp.float32),
            pltpu.VMEM((CH, HID), jnp.float32),
            pltpu.VMEM((2, HID), jnp.float32),
            pltpu.VMEM((3, HID), jnp.float32),
            pltpu.VMEM((6 * HID,), jnp.float32),
            pltpu.VMEM((HID,), jnp.float32),
            pltpu.VMEM((HID,), jnp.float32),
            pltpu.SemaphoreType.DMA,
        ],
    )
    return k(ids, tt, at, word_emb, pos_emb, type_emb, tag_emb, gamma, beta)


def kernel(input_ids, token_type_ids, answer_tag_ids, word_emb, pos_emb,
           type_emb, tag_emb, ln_gamma, ln_beta):
    b, s = input_ids.shape
    ids = input_ids.reshape(-1).astype(jnp.int32)
    tt = token_type_ids.reshape(-1).astype(jnp.int32)
    at = answer_tag_ids.reshape(-1).astype(jnp.int32)
    out = _run(ids, tt, at, word_emb, pos_emb, type_emb, tag_emb,
               ln_gamma, ln_beta)
    return out.reshape(b, s, HID)


# speedup vs baseline: 4.5507x; 4.5507x over previous
import functools

import jax
import jax.numpy as jnp
from jax import lax
from jax.experimental import pallas as pl
from jax.experimental.pallas import tpu as pltpu
from jax.experimental.pallas import tpu_sc as plsc

HID = 768
EPS = 1e-12
NC, NS, L = 2, 16, 16
NW = NC * NS
GCH = 64
BLK = 1024


def _gather_body(ids_hbm, word_hbm, out_hbm, idx_v, rows, sem_g0, sem_g1,
                 sem_s0, sem_s1, sem_i):
    tok = ids_hbm.shape[0]
    tpw = tok // NW
    n_ch = tpw // GCH

    wid = lax.axis_index("s") * NC + lax.axis_index("c")
    base = wid * tpw

    pltpu.async_copy(ids_hbm.at[pl.ds(base, tpw)], idx_v, sem_i).wait()

    gsems = [sem_g0, sem_g1]
    ssems = [sem_s0, sem_s1]
    copies = [None, None]
    scats = [None, None]
    for g in range(n_ch):
        slot = g % 2
        if scats[slot] is not None:
            scats[slot].wait()
        copies[slot] = pltpu.async_copy(
            word_hbm.at[idx_v.at[pl.ds(g * GCH, GCH)]],
            rows.at[pl.ds(slot * GCH, GCH)], gsems[slot])
        if g > 0:
            pslot = (g - 1) % 2
            copies[pslot].wait()
            scats[pslot] = pltpu.async_copy(
                rows.at[pl.ds(pslot * GCH, GCH)],
                out_hbm.at[pl.ds(base + (g - 1) * GCH, GCH)], ssems[pslot])
    last = n_ch - 1
    copies[last % 2].wait()
    scats[last % 2] = pltpu.async_copy(
        rows.at[pl.ds((last % 2) * GCH, GCH)],
        out_hbm.at[pl.ds(base + last * GCH, GCH)], ssems[last % 2])
    scats[(last - 1) % 2].wait()
    scats[last % 2].wait()


@jax.jit
def _sc_gather(ids, word_emb):
    tok = ids.shape[0]
    mesh = plsc.VectorSubcoreMesh(core_axis_name="c", subcore_axis_name="s")
    k = pl.kernel(
        _gather_body,
        out_type=jax.ShapeDtypeStruct((tok, HID), jnp.float32),
        mesh=mesh,
        scratch_types=[
            pltpu.VMEM((tok // NW,), jnp.int32),
            pltpu.VMEM((2 * GCH, HID), jnp.float32),
            pltpu.SemaphoreType.DMA,
            pltpu.SemaphoreType.DMA,
            pltpu.SemaphoreType.DMA,
            pltpu.SemaphoreType.DMA,
            pltpu.SemaphoreType.DMA,
        ],
    )
    return k(ids, word_emb)


def _ln_body(words_ref, pos_ref, tt_ref, at_ref, type_ref, tag_ref,
             gam_ref, bet_ref, out_ref):
    w = words_ref[...]
    p = pos_ref[...]
    tt = tt_ref[0]
    at = at_ref[0]
    t_rows = jnp.where(tt == 0, type_ref[0][None, :], type_ref[1][None, :])
    a_rows = jnp.where(at == 0, tag_ref[0][None, :],
                       jnp.where(at == 1, tag_ref[1][None, :],
                                 tag_ref[2][None, :]))
    e = w + p + t_rows + a_rows
    mean = jnp.mean(e, axis=1, keepdims=True)
    var = jnp.mean(e * e, axis=1, keepdims=True) - mean * mean
    normed = (e - mean) * lax.rsqrt(var + EPS)
    out_ref[...] = normed * gam_ref[0][None, :] + bet_ref[0][None, :]


@jax.jit
def _tc_ln(words, pos_emb, tt3, at3, type_emb, tag_emb, gamma, beta):
    tok = words.shape[0]
    seq = pos_emb.shape[0]
    n_s = seq // BLK
    n_b = tok // seq
    grid = (n_s, n_b)

    return pl.pallas_call(
        _ln_body,
        grid=grid,
        in_specs=[
            pl.BlockSpec((BLK, HID), lambda j, b: (b * n_s + j, 0)),
            pl.BlockSpec((BLK, HID), lambda j, b: (j, 0)),
            pl.BlockSpec((1, BLK, 1), lambda j, b: (b * n_s + j, 0, 0)),
            pl.BlockSpec((1, BLK, 1), lambda j, b: (b * n_s + j, 0, 0)),
            pl.BlockSpec((2, HID), lambda j, b: (0, 0)),
            pl.BlockSpec((3, HID), lambda j, b: (0, 0)),
            pl.BlockSpec((1, HID), lambda j, b: (0, 0)),
            pl.BlockSpec((1, HID), lambda j, b: (0, 0)),
        ],
        out_specs=pl.BlockSpec((BLK, HID), lambda j, b: (b * n_s + j, 0)),
        out_shape=jax.ShapeDtypeStruct((tok, HID), jnp.float32),
    )(words, pos_emb, tt3, at3, type_emb, tag_emb, gamma, beta)


def kernel(input_ids, token_type_ids, answer_tag_ids, word_emb, pos_emb,
           type_emb, tag_emb, ln_gamma, ln_beta):
    b, s = input_ids.shape
    tok = b * s
    ids = input_ids.reshape(-1).astype(jnp.int32)
    tt3 = token_type_ids.astype(jnp.int32).reshape(tok // BLK, BLK, 1)
    at3 = answer_tag_ids.astype(jnp.int32).reshape(tok // BLK, BLK, 1)
    words = _sc_gather(ids, word_emb)
    out = _tc_ln(words, pos_emb, tt3, at3, type_emb, tag_emb,
                 ln_gamma.reshape(1, HID), ln_beta.reshape(1, HID))
    return out.reshape(b, s, HID)
